# Initial kernel scaffold; baseline (speedup 1.0000x reference)
import jax
import jax.numpy as jnp
from jax.experimental import pallas as pl


def kernel(x, edge_index, W1, b1, g1, be1, W2, b2, g2, be2, W3, b3, g3, be3, Wt, bt, Wf1, bf1, Wf2, bf2):
    raise NotImplementedError("write your pallas kernel here")


# speedup vs baseline: 8.4392x; 8.4392x over previous
import functools

import jax
import jax.numpy as jnp
from jax import lax
from jax.experimental import pallas as pl
from jax.experimental.pallas import tpu as pltpu
from jax.experimental.pallas import tpu_sc as plsc

N_NODES = 10000
N_HE = 10000
N_INC = 320000
EPS = 1e-5
TOPO_W = 2.0

NCORE = 2
NSUB = 16
NW = NCORE * NSUB
CHUNK = 128
NCHUNK = -(-N_INC // (NW * CHUNK))
TOT_INC = NW * CHUNK * NCHUNK
ZROWS = 632
ACC_ROWS = ZROWS * NSUB
OROWS = ZROWS
CNT_W = 16


def _make_seg_kernel(F):
    mesh = plsc.VectorSubcoreMesh(core_axis_name="c", subcore_axis_name="s")

    @functools.partial(
        pl.kernel,
        mesh=mesh,
        out_type=jax.ShapeDtypeStruct((NCORE, ACC_ROWS, F), jnp.float32),
        compiler_params=pltpu.CompilerParams(use_tc_tiling_on_sc=False),
        scratch_types=[
            pltpu.VMEM((NCHUNK, CHUNK), jnp.int32),
            pltpu.VMEM((NCHUNK, CHUNK), jnp.int32),
            pltpu.VMEM((CHUNK, F), jnp.float32),
            pltpu.VMEM_SHARED((ACC_ROWS, F), jnp.float32),
        ],
    )
    def seg(table_hbm, src_hbm, dst_hbm, zeros_hbm, out_hbm,
            src_v, dst_v, rows_v, acc):
        c = lax.axis_index("c")
        s = lax.axis_index("s")
        wid = c * NSUB + s
        pltpu.sync_copy(zeros_hbm.at[pl.ds(s * ZROWS, ZROWS)],
                        acc.at[pl.ds(s * ZROWS, ZROWS)])
        pltpu.sync_copy(src_hbm.at[wid], src_v)
        pltpu.sync_copy(dst_hbm.at[wid], dst_v)
        plsc.subcore_barrier()

        def body(k, carry):
            pltpu.sync_copy(table_hbm.at[src_v.at[k]], rows_v)
            pltpu.sync_copy(rows_v, acc.at[dst_v.at[k]], add=True)
            return carry

        lax.fori_loop(0, NCHUNK, body, 0)
        plsc.subcore_barrier()
        pltpu.sync_copy(acc.at[pl.ds(s * OROWS, OROWS)],
                        out_hbm.at[c, pl.ds(s * OROWS, OROWS)])

    return seg


def _make_cnt_kernel():
    mesh = plsc.VectorSubcoreMesh(core_axis_name="c", subcore_axis_name="s")

    @functools.partial(
        pl.kernel,
        mesh=mesh,
        out_type=jax.ShapeDtypeStruct((NCORE, 2, ACC_ROWS, CNT_W), jnp.float32),
        compiler_params=pltpu.CompilerParams(use_tc_tiling_on_sc=False),
        scratch_types=[
            pltpu.VMEM((NCHUNK, CHUNK), jnp.int32),
            pltpu.VMEM((NCHUNK, CHUNK), jnp.int32),
            pltpu.VMEM((CHUNK, CNT_W), jnp.float32),
            pltpu.VMEM_SHARED((ACC_ROWS, CNT_W), jnp.float32),
            pltpu.VMEM_SHARED((ACC_ROWS, CNT_W), jnp.float32),
        ],
    )
    def cnt(nidx_hbm, eidx_hbm, ones_hbm, zeros_hbm, out_hbm,
            nidx_v, eidx_v, ones_v, accn, acce):
        c = lax.axis_index("c")
        s = lax.axis_index("s")
        wid = c * NSUB + s
        pltpu.sync_copy(zeros_hbm.at[pl.ds(s * ZROWS, ZROWS)],
                        accn.at[pl.ds(s * ZROWS, ZROWS)])
        pltpu.sync_copy(zeros_hbm.at[pl.ds(s * ZROWS, ZROWS)],
                        acce.at[pl.ds(s * ZROWS, ZROWS)])
        pltpu.sync_copy(ones_hbm, ones_v)
        pltpu.sync_copy(nidx_hbm.at[wid], nidx_v)
        pltpu.sync_copy(eidx_hbm.at[wid], eidx_v)
        plsc.subcore_barrier()

        def body(k, carry):
            pltpu.sync_copy(ones_v, accn.at[nidx_v.at[k]], add=True)
            pltpu.sync_copy(ones_v, acce.at[eidx_v.at[k]], add=True)
            return carry

        lax.fori_loop(0, NCHUNK, body, 0)
        plsc.subcore_barrier()
        pltpu.sync_copy(accn.at[pl.ds(s * OROWS, OROWS)],
                        out_hbm.at[c, 0, pl.ds(s * OROWS, OROWS)])
        pltpu.sync_copy(acce.at[pl.ds(s * OROWS, OROWS)],
                        out_hbm.at[c, 1, pl.ds(s * OROWS, OROWS)])

    return cnt


def _mm_body(x_ref, w_ref, o_ref):
    o_ref[...] = jnp.dot(x_ref[...], w_ref[...],
                         preferred_element_type=jnp.float32)


def _tc_mm(x, w):
    return pl.pallas_call(
        _mm_body,
        out_shape=jax.ShapeDtypeStruct((x.shape[0], w.shape[1]), jnp.float32),
    )(x, w)


def _scale_body(p_ref, cb0_ref, cb1_ref, o_ref):
    b = (cb0_ref[...] + cb1_ref[...])[:, 0:1]
    binv = jnp.where(b > 0, 1.0 / b, 0.0)
    o_ref[...] = binv * (p_ref[0] + p_ref[1])


def _tc_scale(p, cb0, cb1):
    return pl.pallas_call(
        _scale_body,
        out_shape=jax.ShapeDtypeStruct(p.shape[1:], jnp.float32),
    )(p, cb0, cb1)


def _post_body(q_ref, cd0_ref, cd1_ref, b_ref, g_ref, be_ref, w_ref, o_ref):
    d = (cd0_ref[...] + cd1_ref[...])[:, 0:1]
    dinv = jnp.where(d > 0, 1.0 / d, 0.0)
    t = dinv * (q_ref[0] + q_ref[1]) + b_ref[...]
    mu = jnp.mean(t, axis=0, keepdims=True)
    var = jnp.mean((t - mu) ** 2, axis=0, keepdims=True)
    h = g_ref[...] * (t - mu) / jnp.sqrt(var + EPS) + be_ref[...]
    h = jnp.maximum(h, 0.0)
    o_ref[...] = jnp.dot(h, w_ref[...], preferred_element_type=jnp.float32)


def _tc_post(q, cd0, cd1, b, g, be, w):
    return pl.pallas_call(
        _post_body,
        out_shape=jax.ShapeDtypeStruct((q.shape[1], w.shape[1]), jnp.float32),
    )(q, cd0, cd1, b.reshape(1, -1), g.reshape(1, -1), be.reshape(1, -1), w)


def _head_body(q_ref, cd0_ref, cd1_ref, b_ref, g_ref, be_ref, bt_ref,
               wf1_ref, bf1_ref, wf2_ref, bf2_ref, o_ref):
    d = (cd0_ref[...] + cd1_ref[...])[:, 0:1]
    dinv = jnp.where(d > 0, 1.0 / d, 0.0)
    t = dinv * (q_ref[0] + q_ref[1]) + b_ref[...]
    mu = jnp.mean(t, axis=0, keepdims=True)
    var = jnp.mean((t - mu) ** 2, axis=0, keepdims=True)
    h = g_ref[...] * (t - mu) / jnp.sqrt(var + EPS) + be_ref[...]
    te2 = jnp.maximum(bt_ref[...], 0.0) * TOPO_W
    row = jnp.dot(te2, wf1_ref[64:128, :],
                  preferred_element_type=jnp.float32)
    o = jnp.dot(h, wf1_ref[0:64, :],
                preferred_element_type=jnp.float32) + row + bf1_ref[...]
    o = jnp.maximum(o, 0.0)
    lg = jnp.dot(o, wf2_ref[...], preferred_element_type=jnp.float32)
    lg = lg + bf2_ref[...]
    m = jnp.max(lg, axis=1, keepdims=True)
    z = lg - m
    lse = jnp.log(jnp.sum(jnp.exp(z), axis=1, keepdims=True))
    o_ref[...] = z - lse


def _tc_head(q, cd0, cd1, b, g, be, bt, wf1, bf1, wf2, bf2):
    return pl.pallas_call(
        _head_body,
        out_shape=jax.ShapeDtypeStruct((q.shape[1], wf2.shape[1]), jnp.float32),
    )(q, cd0, cd1, b.reshape(1, -1), g.reshape(1, -1), be.reshape(1, -1),
      bt.reshape(1, -1), wf1, bf1.reshape(1, -1), wf2, bf2.reshape(1, -1))


def kernel(x, edge_index, W1, b1, g1, be1, W2, b2, g2, be2, W3, b3, g3, be3,
           Wt, bt, Wf1, bf1, Wf2, bf2):
    node = edge_index[0].astype(jnp.int32)
    he = edge_index[1].astype(jnp.int32)
    pad = TOT_INC - N_INC
    node_src = jnp.concatenate([node, jnp.zeros((pad,), jnp.int32)])
    node_dst = jnp.concatenate([node, jnp.full((pad,), N_NODES, jnp.int32)])
    he_src = jnp.concatenate([he, jnp.zeros((pad,), jnp.int32)])
    he_dst = jnp.concatenate([he, jnp.full((pad,), N_NODES, jnp.int32)])
    shp = (NW, NCHUNK, CHUNK)
    node_src, node_dst = node_src.reshape(shp), node_dst.reshape(shp)
    he_src, he_dst = he_src.reshape(shp), he_dst.reshape(shp)

    z64 = jnp.zeros((ACC_ROWS, 64), jnp.float32)
    z128 = jnp.zeros((ACC_ROWS, 128), jnp.float32)
    zc = jnp.zeros((ACC_ROWS, CNT_W), jnp.float32)
    ones = jnp.ones((CHUNK, CNT_W), jnp.float32)

    seg64 = _make_seg_kernel(64)
    seg128 = _make_seg_kernel(128)
    cntk = _make_cnt_kernel()

    cnt = cntk(node_dst, he_dst, ones, zc)
    cd0, cd1 = cnt[0, 0, :N_NODES], cnt[1, 0, :N_NODES]
    cb0, cb1 = cnt[0, 1, :N_NODES], cnt[1, 1, :N_NODES]

    def unpad(a):
        return a[:, :N_NODES]

    xw = _tc_mm(x, W1)
    p = unpad(seg64(xw, node_src, he_dst, z64))
    t = _tc_scale(p, cb0, cb1)
    q = unpad(seg64(t, he_src, node_dst, z64))
    xw = _tc_post(q, cd0, cd1, b1, g1, be1, W2)

    p = unpad(seg128(xw, node_src, he_dst, z128))
    t = _tc_scale(p, cb0, cb1)
    q = unpad(seg128(t, he_src, node_dst, z128))
    xw = _tc_post(q, cd0, cd1, b2, g2, be2, W3)

    p = unpad(seg64(xw, node_src, he_dst, z64))
    t = _tc_scale(p, cb0, cb1)
    q = unpad(seg64(t, he_src, node_dst, z64))

    return _tc_head(q, cd0, cd1, b3, g3, be3, bt, Wf1, bf1, Wf2, bf2)


# speedup vs baseline: 10.1633x; 1.2043x over previous
import functools

import jax
import jax.numpy as jnp
from jax import lax
from jax.experimental import pallas as pl
from jax.experimental.pallas import tpu as pltpu
from jax.experimental.pallas import tpu_sc as plsc

N_NODES = 10000
N_HE = 10000
N_INC = 320000
EPS = 1e-5
TOPO_W = 2.0

NCORE = 2
NSUB = 16
NW = NCORE * NSUB
CHUNK = 128
NCHUNK = -(-N_INC // (NW * CHUNK))
CHUNK_W = 96
NCHUNK_W = -(-N_INC // (NW * CHUNK_W))
ZROWS = 632
ACC_ROWS = ZROWS * NSUB
OROWS = ZROWS
CNT_W = 16


def _make_seg_kernel(F, chunk, nchunk):
    mesh = plsc.VectorSubcoreMesh(core_axis_name="c", subcore_axis_name="s")

    @functools.partial(
        pl.kernel,
        mesh=mesh,
        out_type=jax.ShapeDtypeStruct((NCORE, ACC_ROWS, F), jnp.float32),
        compiler_params=pltpu.CompilerParams(use_tc_tiling_on_sc=False),
        scratch_types=[
            pltpu.VMEM((nchunk, chunk), jnp.int32),
            pltpu.VMEM((nchunk, chunk), jnp.int32),
            pltpu.VMEM((chunk, F), jnp.float32),
            pltpu.VMEM((chunk, F), jnp.float32),
            pltpu.VMEM_SHARED((ACC_ROWS, F), jnp.float32),
            pltpu.SemaphoreType.DMA,
            pltpu.SemaphoreType.DMA,
        ],
    )
    def seg(table_hbm, src_hbm, dst_hbm, zeros_hbm, out_hbm,
            src_v, dst_v, rows_a, rows_b, acc, gsem_a, gsem_b):
        c = lax.axis_index("c")
        s = lax.axis_index("s")
        wid = c * NSUB + s
        pltpu.sync_copy(zeros_hbm.at[pl.ds(s * ZROWS, ZROWS)],
                        acc.at[pl.ds(s * ZROWS, ZROWS)])
        pltpu.sync_copy(src_hbm.at[wid], src_v)
        pltpu.sync_copy(dst_hbm.at[wid], dst_v)
        plsc.subcore_barrier()

        pltpu.async_copy(table_hbm.at[src_v.at[0]], rows_a, gsem_a)

        def body(j, carry):
            k0 = 2 * j
            pltpu.make_async_copy(table_hbm.at[src_v.at[k0]],
                                  rows_a, gsem_a).wait()

            @pl.when(k0 + 1 < nchunk)
            def _():
                pltpu.async_copy(table_hbm.at[src_v.at[k0 + 1]],
                                 rows_b, gsem_b)

            pltpu.sync_copy(rows_a, acc.at[dst_v.at[k0]], add=True)

            @pl.when(k0 + 1 < nchunk)
            def _():
                pltpu.make_async_copy(table_hbm.at[src_v.at[k0 + 1]],
                                      rows_b, gsem_b).wait()

                @pl.when(k0 + 2 < nchunk)
                def _():
                    pltpu.async_copy(table_hbm.at[src_v.at[k0 + 2]],
                                     rows_a, gsem_a)

                pltpu.sync_copy(rows_b, acc.at[dst_v.at[k0 + 1]], add=True)

            return carry

        lax.fori_loop(0, (nchunk + 1) // 2, body, 0)
        plsc.subcore_barrier()
        pltpu.sync_copy(acc.at[pl.ds(s * OROWS, OROWS)],
                        out_hbm.at[c, pl.ds(s * OROWS, OROWS)])

    return seg


def _make_cnt_kernel():
    mesh = plsc.VectorSubcoreMesh(core_axis_name="c", subcore_axis_name="s")

    @functools.partial(
        pl.kernel,
        mesh=mesh,
        out_type=jax.ShapeDtypeStruct((NCORE, 2, ACC_ROWS, CNT_W), jnp.float32),
        compiler_params=pltpu.CompilerParams(use_tc_tiling_on_sc=False),
        scratch_types=[
            pltpu.VMEM((NCHUNK, CHUNK), jnp.int32),
            pltpu.VMEM((NCHUNK, CHUNK), jnp.int32),
            pltpu.VMEM((CHUNK, CNT_W), jnp.float32),
            pltpu.VMEM_SHARED((ACC_ROWS, CNT_W), jnp.float32),
            pltpu.VMEM_SHARED((ACC_ROWS, CNT_W), jnp.float32),
        ],
    )
    def cnt(nidx_hbm, eidx_hbm, ones_hbm, zeros_hbm, out_hbm,
            nidx_v, eidx_v, ones_v, accn, acce):
        c = lax.axis_index("c")
        s = lax.axis_index("s")
        wid = c * NSUB + s
        pltpu.sync_copy(zeros_hbm.at[pl.ds(s * ZROWS, ZROWS)],
                        accn.at[pl.ds(s * ZROWS, ZROWS)])
        pltpu.sync_copy(zeros_hbm.at[pl.ds(s * ZROWS, ZROWS)],
                        acce.at[pl.ds(s * ZROWS, ZROWS)])
        pltpu.sync_copy(ones_hbm, ones_v)
        pltpu.sync_copy(nidx_hbm.at[wid], nidx_v)
        pltpu.sync_copy(eidx_hbm.at[wid], eidx_v)
        plsc.subcore_barrier()

        def body(k, carry):
            pltpu.sync_copy(ones_v, accn.at[nidx_v.at[k]], add=True)
            pltpu.sync_copy(ones_v, acce.at[eidx_v.at[k]], add=True)
            return carry

        lax.fori_loop(0, NCHUNK, body, 0)
        plsc.subcore_barrier()
        pltpu.sync_copy(accn.at[pl.ds(s * OROWS, OROWS)],
                        out_hbm.at[c, 0, pl.ds(s * OROWS, OROWS)])
        pltpu.sync_copy(acce.at[pl.ds(s * OROWS, OROWS)],
                        out_hbm.at[c, 1, pl.ds(s * OROWS, OROWS)])

    return cnt


def _mm_body(x_ref, w_ref, o_ref):
    o_ref[...] = jnp.dot(x_ref[...], w_ref[...],
                         preferred_element_type=jnp.float32)


def _tc_mm(x, w):
    return pl.pallas_call(
        _mm_body,
        out_shape=jax.ShapeDtypeStruct((x.shape[0], w.shape[1]), jnp.float32),
    )(x, w)


def _scale_body(p_ref, cb0_ref, cb1_ref, o_ref):
    b = (cb0_ref[...] + cb1_ref[...])[:, 0:1]
    binv = jnp.where(b > 0, 1.0 / b, 0.0)
    o_ref[...] = binv * (p_ref[0] + p_ref[1])


def _tc_scale(p, cb0, cb1):
    return pl.pallas_call(
        _scale_body,
        out_shape=jax.ShapeDtypeStruct(p.shape[1:], jnp.float32),
    )(p, cb0, cb1)


def _post_body(q_ref, cd0_ref, cd1_ref, b_ref, g_ref, be_ref, w_ref, o_ref):
    d = (cd0_ref[...] + cd1_ref[...])[:, 0:1]
    dinv = jnp.where(d > 0, 1.0 / d, 0.0)
    t = dinv * (q_ref[0] + q_ref[1]) + b_ref[...]
    mu = jnp.mean(t, axis=0, keepdims=True)
    var = jnp.mean((t - mu) ** 2, axis=0, keepdims=True)
    h = g_ref[...] * (t - mu) / jnp.sqrt(var + EPS) + be_ref[...]
    h = jnp.maximum(h, 0.0)
    o_ref[...] = jnp.dot(h, w_ref[...], preferred_element_type=jnp.float32)


def _tc_post(q, cd0, cd1, b, g, be, w):
    return pl.pallas_call(
        _post_body,
        out_shape=jax.ShapeDtypeStruct((q.shape[1], w.shape[1]), jnp.float32),
    )(q, cd0, cd1, b.reshape(1, -1), g.reshape(1, -1), be.reshape(1, -1), w)


def _head_body(q_ref, cd0_ref, cd1_ref, b_ref, g_ref, be_ref, bt_ref,
               wf1_ref, bf1_ref, wf2_ref, bf2_ref, o_ref):
    d = (cd0_ref[...] + cd1_ref[...])[:, 0:1]
    dinv = jnp.where(d > 0, 1.0 / d, 0.0)
    t = dinv * (q_ref[0] + q_ref[1]) + b_ref[...]
    mu = jnp.mean(t, axis=0, keepdims=True)
    var = jnp.mean((t - mu) ** 2, axis=0, keepdims=True)
    h = g_ref[...] * (t - mu) / jnp.sqrt(var + EPS) + be_ref[...]
    te2 = jnp.maximum(bt_ref[...], 0.0) * TOPO_W
    row = jnp.dot(te2, wf1_ref[64:128, :],
                  preferred_element_type=jnp.float32)
    o = jnp.dot(h, wf1_ref[0:64, :],
                preferred_element_type=jnp.float32) + row + bf1_ref[...]
    o = jnp.maximum(o, 0.0)
    lg = jnp.dot(o, wf2_ref[...], preferred_element_type=jnp.float32)
    lg = lg + bf2_ref[...]
    m = jnp.max(lg, axis=1, keepdims=True)
    z = lg - m
    lse = jnp.log(jnp.sum(jnp.exp(z), axis=1, keepdims=True))
    o_ref[...] = z - lse


def _tc_head(q, cd0, cd1, b, g, be, bt, wf1, bf1, wf2, bf2):
    return pl.pallas_call(
        _head_body,
        out_shape=jax.ShapeDtypeStruct((q.shape[1], wf2.shape[1]), jnp.float32),
    )(q, cd0, cd1, b.reshape(1, -1), g.reshape(1, -1), be.reshape(1, -1),
      bt.reshape(1, -1), wf1, bf1.reshape(1, -1), wf2, bf2.reshape(1, -1))


def kernel(x, edge_index, W1, b1, g1, be1, W2, b2, g2, be2, W3, b3, g3, be3,
           Wt, bt, Wf1, bf1, Wf2, bf2):
    node = edge_index[0].astype(jnp.int32)
    he = edge_index[1].astype(jnp.int32)

    def layout(idx, padval, chunk, nchunk):
        pad = NW * chunk * nchunk - N_INC
        full = jnp.concatenate([idx, jnp.full((pad,), padval, jnp.int32)])
        return full.reshape(NW, nchunk, chunk)

    node_src = layout(node, 0, CHUNK, NCHUNK)
    node_dst = layout(node, N_NODES, CHUNK, NCHUNK)
    he_src = layout(he, 0, CHUNK, NCHUNK)
    he_dst = layout(he, N_NODES, CHUNK, NCHUNK)
    node_src_w = layout(node, 0, CHUNK_W, NCHUNK_W)
    node_dst_w = layout(node, N_NODES, CHUNK_W, NCHUNK_W)
    he_src_w = layout(he, 0, CHUNK_W, NCHUNK_W)
    he_dst_w = layout(he, N_NODES, CHUNK_W, NCHUNK_W)

    z64 = jnp.zeros((ACC_ROWS, 64), jnp.float32)
    z128 = jnp.zeros((ACC_ROWS, 128), jnp.float32)
    zc = jnp.zeros((ACC_ROWS, CNT_W), jnp.float32)
    ones = jnp.ones((CHUNK, CNT_W), jnp.float32)

    seg64 = _make_seg_kernel(64, CHUNK, NCHUNK)
    seg128 = _make_seg_kernel(128, CHUNK_W, NCHUNK_W)
    cntk = _make_cnt_kernel()

    cnt = cntk(node_dst, he_dst, ones, zc)
    cd0, cd1 = cnt[0, 0, :N_NODES], cnt[1, 0, :N_NODES]
    cb0, cb1 = cnt[0, 1, :N_NODES], cnt[1, 1, :N_NODES]

    def unpad(a):
        return a[:, :N_NODES]

    xw = _tc_mm(x, W1)
    p = unpad(seg64(xw, node_src, he_dst, z64))
    t = _tc_scale(p, cb0, cb1)
    q = unpad(seg64(t, he_src, node_dst, z64))
    xw = _tc_post(q, cd0, cd1, b1, g1, be1, W2)

    p = unpad(seg128(xw, node_src_w, he_dst_w, z128))
    t = _tc_scale(p, cb0, cb1)
    q = unpad(seg128(t, he_src_w, node_dst_w, z128))
    xw = _tc_post(q, cd0, cd1, b2, g2, be2, W3)

    p = unpad(seg64(xw, node_src, he_dst, z64))
    t = _tc_scale(p, cb0, cb1)
    q = unpad(seg64(t, he_src, node_dst, z64))

    return _tc_head(q, cd0, cd1, b3, g3, be3, bt, Wf1, bf1, Wf2, bf2)


# speedup vs baseline: 12.6437x; 1.2441x over previous
import functools

import jax
import jax.numpy as jnp
from jax import lax
from jax.experimental import pallas as pl
from jax.experimental.pallas import tpu as pltpu
from jax.experimental.pallas import tpu_sc as plsc

N_NODES = 10000
N_HE = 10000
N_INC = 320000
EPS = 1e-5
TOPO_W = 2.0

NCORE = 2
NSUB = 16
NW = NCORE * NSUB
CHUNK = 128
NCHUNK = -(-N_INC // (NW * CHUNK))
CHUNK_W = 64
NCHUNK_W = -(-N_INC // (NW * CHUNK_W))
ZROWS = 632
ACC_ROWS = ZROWS * NSUB
OROWS = ZROWS
CNT_W = 16


def _make_seg_kernel(F, chunk, nchunk, depth):
    mesh = plsc.VectorSubcoreMesh(core_axis_name="c", subcore_axis_name="s")
    ngroups = -(-nchunk // depth)

    @functools.partial(
        pl.kernel,
        mesh=mesh,
        out_type=jax.ShapeDtypeStruct((NCORE, ACC_ROWS, F), jnp.float32),
        compiler_params=pltpu.CompilerParams(use_tc_tiling_on_sc=False),
        scratch_types=[
            pltpu.VMEM((nchunk, chunk), jnp.int32),
            pltpu.VMEM((nchunk, chunk), jnp.int32),
        ] + [pltpu.VMEM((chunk, F), jnp.float32)] * depth + [
            pltpu.VMEM_SHARED((ACC_ROWS, F), jnp.float32),
        ] + [pltpu.SemaphoreType.DMA] * (2 * depth),
    )
    def seg(table_hbm, src_hbm, dst_hbm, zeros_hbm, out_hbm,
            src_v, dst_v, *rest):
        rows = rest[:depth]
        acc = rest[depth]
        gsems = rest[depth + 1:2 * depth + 1]
        ssems = rest[2 * depth + 1:3 * depth + 1]
        c = lax.axis_index("c")
        s = lax.axis_index("s")
        wid = c * NSUB + s
        pltpu.sync_copy(zeros_hbm.at[pl.ds(s * ZROWS, ZROWS)],
                        acc.at[pl.ds(s * ZROWS, ZROWS)])
        pltpu.sync_copy(src_hbm.at[wid], src_v)
        pltpu.sync_copy(dst_hbm.at[wid], dst_v)
        plsc.subcore_barrier()

        for b in range(depth - 1):
            pltpu.async_copy(table_hbm.at[src_v.at[b]], rows[b], gsems[b])

        def body(g, carry):
            kb = g * depth
            for b in range(depth):
                k = kb + b
                bprev = (b - 1) % depth

                @pl.when(k < nchunk)
                def _(k=k, b=b, bprev=bprev):
                    pltpu.make_async_copy(table_hbm.at[src_v.at[k]],
                                          rows[b], gsems[b]).wait()
                    pltpu.async_copy(rows[b], acc.at[dst_v.at[k]],
                                     ssems[b], add=True)

                    @pl.when(k + depth - 1 < nchunk)
                    def _():
                        @pl.when(k > 0)
                        def _():
                            pltpu.make_async_copy(
                                rows[bprev], acc.at[dst_v.at[0]],
                                ssems[bprev]).wait()
                        pltpu.async_copy(table_hbm.at[src_v.at[k + depth - 1]],
                                         rows[bprev], gsems[bprev])
            return carry

        lax.fori_loop(0, ngroups, body, 0)
        for b in range(depth):
            pltpu.make_async_copy(rows[b], acc.at[dst_v.at[0]],
                                  ssems[b]).wait()
        plsc.subcore_barrier()
        pltpu.sync_copy(acc.at[pl.ds(s * OROWS, OROWS)],
                        out_hbm.at[c, pl.ds(s * OROWS, OROWS)])

    return seg


def _make_cnt_kernel():
    mesh = plsc.VectorSubcoreMesh(core_axis_name="c", subcore_axis_name="s")

    @functools.partial(
        pl.kernel,
        mesh=mesh,
        out_type=jax.ShapeDtypeStruct((NCORE, 2, ACC_ROWS, CNT_W), jnp.float32),
        compiler_params=pltpu.CompilerParams(use_tc_tiling_on_sc=False),
        scratch_types=[
            pltpu.VMEM((NCHUNK, CHUNK), jnp.int32),
            pltpu.VMEM((NCHUNK, CHUNK), jnp.int32),
            pltpu.VMEM((CHUNK, CNT_W), jnp.float32),
            pltpu.VMEM_SHARED((ACC_ROWS, CNT_W), jnp.float32),
            pltpu.VMEM_SHARED((ACC_ROWS, CNT_W), jnp.float32),
            pltpu.SemaphoreType.DMA,
            pltpu.SemaphoreType.DMA,
        ],
    )
    def cnt(nidx_hbm, eidx_hbm, ones_hbm, zeros_hbm, out_hbm,
            nidx_v, eidx_v, ones_v, accn, acce, sem_n, sem_e):
        c = lax.axis_index("c")
        s = lax.axis_index("s")
        wid = c * NSUB + s
        pltpu.sync_copy(zeros_hbm.at[pl.ds(s * ZROWS, ZROWS)],
                        accn.at[pl.ds(s * ZROWS, ZROWS)])
        pltpu.sync_copy(zeros_hbm.at[pl.ds(s * ZROWS, ZROWS)],
                        acce.at[pl.ds(s * ZROWS, ZROWS)])
        pltpu.sync_copy(ones_hbm, ones_v)
        pltpu.sync_copy(nidx_hbm.at[wid], nidx_v)
        pltpu.sync_copy(eidx_hbm.at[wid], eidx_v)
        plsc.subcore_barrier()

        def body(k, carry):
            @pl.when(k > 0)
            def _():
                pltpu.make_async_copy(ones_v, accn.at[nidx_v.at[0]],
                                      sem_n).wait()
                pltpu.make_async_copy(ones_v, acce.at[eidx_v.at[0]],
                                      sem_e).wait()
            pltpu.async_copy(ones_v, accn.at[nidx_v.at[k]], sem_n, add=True)
            pltpu.async_copy(ones_v, acce.at[eidx_v.at[k]], sem_e, add=True)
            return carry

        lax.fori_loop(0, NCHUNK, body, 0)
        pltpu.make_async_copy(ones_v, accn.at[nidx_v.at[0]], sem_n).wait()
        pltpu.make_async_copy(ones_v, acce.at[eidx_v.at[0]], sem_e).wait()
        plsc.subcore_barrier()
        pltpu.sync_copy(accn.at[pl.ds(s * OROWS, OROWS)],
                        out_hbm.at[c, 0, pl.ds(s * OROWS, OROWS)])
        pltpu.sync_copy(acce.at[pl.ds(s * OROWS, OROWS)],
                        out_hbm.at[c, 1, pl.ds(s * OROWS, OROWS)])

    return cnt


def _mm_body(x_ref, w_ref, o_ref):
    o_ref[...] = jnp.dot(x_ref[...], w_ref[...],
                         preferred_element_type=jnp.float32)


def _tc_mm(x, w):
    return pl.pallas_call(
        _mm_body,
        out_shape=jax.ShapeDtypeStruct((x.shape[0], w.shape[1]), jnp.float32),
    )(x, w)


def _scale_body(p_ref, cb0_ref, cb1_ref, o_ref):
    b = (cb0_ref[...] + cb1_ref[...])[:, 0:1]
    binv = jnp.where(b > 0, 1.0 / b, 0.0)
    o_ref[...] = binv * (p_ref[0] + p_ref[1])


def _tc_scale(p, cb0, cb1):
    return pl.pallas_call(
        _scale_body,
        out_shape=jax.ShapeDtypeStruct(p.shape[1:], jnp.float32),
    )(p, cb0, cb1)


def _post_body(q_ref, cd0_ref, cd1_ref, b_ref, g_ref, be_ref, w_ref, o_ref):
    d = (cd0_ref[...] + cd1_ref[...])[:, 0:1]
    dinv = jnp.where(d > 0, 1.0 / d, 0.0)
    t = dinv * (q_ref[0] + q_ref[1]) + b_ref[...]
    mu = jnp.mean(t, axis=0, keepdims=True)
    var = jnp.mean((t - mu) ** 2, axis=0, keepdims=True)
    h = g_ref[...] * (t - mu) / jnp.sqrt(var + EPS) + be_ref[...]
    h = jnp.maximum(h, 0.0)
    o_ref[...] = jnp.dot(h, w_ref[...], preferred_element_type=jnp.float32)


def _tc_post(q, cd0, cd1, b, g, be, w):
    return pl.pallas_call(
        _post_body,
        out_shape=jax.ShapeDtypeStruct((q.shape[1], w.shape[1]), jnp.float32),
    )(q, cd0, cd1, b.reshape(1, -1), g.reshape(1, -1), be.reshape(1, -1), w)


def _head_body(q_ref, cd0_ref, cd1_ref, b_ref, g_ref, be_ref, bt_ref,
               wf1_ref, bf1_ref, wf2_ref, bf2_ref, o_ref):
    d = (cd0_ref[...] + cd1_ref[...])[:, 0:1]
    dinv = jnp.where(d > 0, 1.0 / d, 0.0)
    t = dinv * (q_ref[0] + q_ref[1]) + b_ref[...]
    mu = jnp.mean(t, axis=0, keepdims=True)
    var = jnp.mean((t - mu) ** 2, axis=0, keepdims=True)
    h = g_ref[...] * (t - mu) / jnp.sqrt(var + EPS) + be_ref[...]
    te2 = jnp.maximum(bt_ref[...], 0.0) * TOPO_W
    row = jnp.dot(te2, wf1_ref[64:128, :],
                  preferred_element_type=jnp.float32)
    o = jnp.dot(h, wf1_ref[0:64, :],
                preferred_element_type=jnp.float32) + row + bf1_ref[...]
    o = jnp.maximum(o, 0.0)
    lg = jnp.dot(o, wf2_ref[...], preferred_element_type=jnp.float32)
    lg = lg + bf2_ref[...]
    m = jnp.max(lg, axis=1, keepdims=True)
    z = lg - m
    lse = jnp.log(jnp.sum(jnp.exp(z), axis=1, keepdims=True))
    o_ref[...] = z - lse


def _tc_head(q, cd0, cd1, b, g, be, bt, wf1, bf1, wf2, bf2):
    return pl.pallas_call(
        _head_body,
        out_shape=jax.ShapeDtypeStruct((q.shape[1], wf2.shape[1]), jnp.float32),
    )(q, cd0, cd1, b.reshape(1, -1), g.reshape(1, -1), be.reshape(1, -1),
      bt.reshape(1, -1), wf1, bf1.reshape(1, -1), wf2, bf2.reshape(1, -1))


def kernel(x, edge_index, W1, b1, g1, be1, W2, b2, g2, be2, W3, b3, g3, be3,
           Wt, bt, Wf1, bf1, Wf2, bf2):
    node = edge_index[0].astype(jnp.int32)
    he = edge_index[1].astype(jnp.int32)

    def layout(idx, padval, chunk, nchunk):
        pad = NW * chunk * nchunk - N_INC
        full = jnp.concatenate([idx, jnp.full((pad,), padval, jnp.int32)])
        return full.reshape(NW, nchunk, chunk)

    node_src = layout(node, 0, CHUNK, NCHUNK)
    node_dst = layout(node, N_NODES, CHUNK, NCHUNK)
    he_src = layout(he, 0, CHUNK, NCHUNK)
    he_dst = layout(he, N_NODES, CHUNK, NCHUNK)
    node_src_w = layout(node, 0, CHUNK_W, NCHUNK_W)
    node_dst_w = layout(node, N_NODES, CHUNK_W, NCHUNK_W)
    he_src_w = layout(he, 0, CHUNK_W, NCHUNK_W)
    he_dst_w = layout(he, N_NODES, CHUNK_W, NCHUNK_W)

    z64 = jnp.zeros((ACC_ROWS, 64), jnp.float32)
    z128 = jnp.zeros((ACC_ROWS, 128), jnp.float32)
    zc = jnp.zeros((ACC_ROWS, CNT_W), jnp.float32)
    ones = jnp.ones((CHUNK, CNT_W), jnp.float32)

    seg64 = _make_seg_kernel(64, CHUNK, NCHUNK, 4)
    seg128 = _make_seg_kernel(128, CHUNK_W, NCHUNK_W, 3)
    cntk = _make_cnt_kernel()

    cnt = cntk(node_dst, he_dst, ones, zc)
    cd0, cd1 = cnt[0, 0, :N_NODES], cnt[1, 0, :N_NODES]
    cb0, cb1 = cnt[0, 1, :N_NODES], cnt[1, 1, :N_NODES]

    def unpad(a):
        return a[:, :N_NODES]

    xw = _tc_mm(x, W1)
    p = unpad(seg64(xw, node_src, he_dst, z64))
    t = _tc_scale(p, cb0, cb1)
    q = unpad(seg64(t, he_src, node_dst, z64))
    xw = _tc_post(q, cd0, cd1, b1, g1, be1, W2)

    p = unpad(seg128(xw, node_src_w, he_dst_w, z128))
    t = _tc_scale(p, cb0, cb1)
    q = unpad(seg128(t, he_src_w, node_dst_w, z128))
    xw = _tc_post(q, cd0, cd1, b2, g2, be2, W3)

    p = unpad(seg64(xw, node_src, he_dst, z64))
    t = _tc_scale(p, cb0, cb1)
    q = unpad(seg64(t, he_src, node_dst, z64))

    return _tc_head(q, cd0, cd1, b3, g3, be3, bt, Wf1, bf1, Wf2, bf2)


# speedup vs baseline: 16.0789x; 1.2717x over previous
import functools

import jax
import jax.numpy as jnp
from jax import lax
from jax.experimental import pallas as pl
from jax.experimental.pallas import tpu as pltpu
from jax.experimental.pallas import tpu_sc as plsc

N_NODES = 10000
N_HE = 10000
N_INC = 320000
EPS = 1e-5
TOPO_W = 2.0

NCORE = 2
NSUB = 16
NW = NCORE * NSUB
CHUNK = 128
NCHUNK = -(-N_INC // (NW * CHUNK))
ZROWS = 632
ACC_ROWS = ZROWS * NSUB
NDUMP = ACC_ROWS - N_NODES
OROWS = ZROWS
CNT_W = 16
FW = 64
DEPTH = 2


def _make_seg_kernel(H):
    mesh = plsc.VectorSubcoreMesh(core_axis_name="c", subcore_axis_name="s")
    ngroups = -(-NCHUNK // DEPTH)

    @functools.partial(
        pl.kernel,
        mesh=mesh,
        out_type=jax.ShapeDtypeStruct((NCORE, H, ACC_ROWS, FW), jnp.float32),
        compiler_params=pltpu.CompilerParams(use_tc_tiling_on_sc=False),
        scratch_types=[
            pltpu.VMEM((NCHUNK, CHUNK), jnp.int32),
            pltpu.VMEM((NCHUNK, CHUNK), jnp.int32),
        ] + [pltpu.VMEM((CHUNK, FW), jnp.float32)] * DEPTH + [
            pltpu.VMEM_SHARED((ACC_ROWS, FW), jnp.float32),
            pltpu.VMEM_SHARED((ACC_ROWS, FW), jnp.float32),
        ] + [pltpu.SemaphoreType.DMA] * (2 * DEPTH),
    )
    def seg(table_hbm, src_hbm, dst_hbm, zeros_hbm, out_hbm,
            src_v, dst_v, *rest):
        rows = rest[:DEPTH]
        tbl = rest[DEPTH]
        acc = rest[DEPTH + 1]
        gsems = rest[DEPTH + 2:2 * DEPTH + 2]
        ssems = rest[2 * DEPTH + 2:3 * DEPTH + 2]
        c = lax.axis_index("c")
        s = lax.axis_index("s")
        wid = c * NSUB + s
        pltpu.sync_copy(src_hbm.at[wid], src_v)
        pltpu.sync_copy(dst_hbm.at[wid], dst_v)

        for h in range(H):
            pltpu.sync_copy(table_hbm.at[h, pl.ds(s * ZROWS, ZROWS)],
                            tbl.at[pl.ds(s * ZROWS, ZROWS)])
            pltpu.sync_copy(zeros_hbm.at[pl.ds(s * ZROWS, ZROWS)],
                            acc.at[pl.ds(s * ZROWS, ZROWS)])
            plsc.subcore_barrier()

            for b in range(DEPTH - 1):
                pltpu.async_copy(tbl.at[src_v.at[b]], rows[b], gsems[b])

            def body(g, carry):
                kb = g * DEPTH
                for b in range(DEPTH):
                    k = kb + b
                    bprev = (b - 1) % DEPTH

                    @pl.when(k < NCHUNK)
                    def _(k=k, b=b, bprev=bprev):
                        pltpu.make_async_copy(tbl.at[src_v.at[k]],
                                              rows[b], gsems[b]).wait()
                        pltpu.async_copy(rows[b], acc.at[dst_v.at[k]],
                                         ssems[b], add=True)

                        @pl.when(k + DEPTH - 1 < NCHUNK)
                        def _():
                            @pl.when(k > 0)
                            def _():
                                pltpu.make_async_copy(
                                    rows[bprev], acc.at[dst_v.at[0]],
                                    ssems[bprev]).wait()
                            pltpu.async_copy(tbl.at[src_v.at[k + DEPTH - 1]],
                                             rows[bprev], gsems[bprev])
                return carry

            lax.fori_loop(0, ngroups, body, 0)
            for b in range(DEPTH):
                pltpu.make_async_copy(rows[b], acc.at[dst_v.at[0]],
                                      ssems[b]).wait()
            plsc.subcore_barrier()
            pltpu.sync_copy(acc.at[pl.ds(s * OROWS, OROWS)],
                            out_hbm.at[c, h, pl.ds(s * OROWS, OROWS)])

    return seg


def _make_cnt_kernel():
    mesh = plsc.VectorSubcoreMesh(core_axis_name="c", subcore_axis_name="s")

    @functools.partial(
        pl.kernel,
        mesh=mesh,
        out_type=jax.ShapeDtypeStruct((NCORE, 2, ACC_ROWS, CNT_W), jnp.float32),
        compiler_params=pltpu.CompilerParams(use_tc_tiling_on_sc=False),
        scratch_types=[
            pltpu.VMEM((NCHUNK, CHUNK), jnp.int32),
            pltpu.VMEM((NCHUNK, CHUNK), jnp.int32),
            pltpu.VMEM((CHUNK, CNT_W), jnp.float32),
            pltpu.VMEM_SHARED((ACC_ROWS, CNT_W), jnp.float32),
            pltpu.VMEM_SHARED((ACC_ROWS, CNT_W), jnp.float32),
            pltpu.SemaphoreType.DMA,
            pltpu.SemaphoreType.DMA,
        ],
    )
    def cnt(nidx_hbm, eidx_hbm, ones_hbm, zeros_hbm, out_hbm,
            nidx_v, eidx_v, ones_v, accn, acce, sem_n, sem_e):
        c = lax.axis_index("c")
        s = lax.axis_index("s")
        wid = c * NSUB + s
        pltpu.sync_copy(zeros_hbm.at[pl.ds(s * ZROWS, ZROWS)],
                        accn.at[pl.ds(s * ZROWS, ZROWS)])
        pltpu.sync_copy(zeros_hbm.at[pl.ds(s * ZROWS, ZROWS)],
                        acce.at[pl.ds(s * ZROWS, ZROWS)])
        pltpu.sync_copy(ones_hbm, ones_v)
        pltpu.sync_copy(nidx_hbm.at[wid], nidx_v)
        pltpu.sync_copy(eidx_hbm.at[wid], eidx_v)
        plsc.subcore_barrier()

        def body(k, carry):
            @pl.when(k > 0)
            def _():
                pltpu.make_async_copy(ones_v, accn.at[nidx_v.at[0]],
                                      sem_n).wait()
                pltpu.make_async_copy(ones_v, acce.at[eidx_v.at[0]],
                                      sem_e).wait()
            pltpu.async_copy(ones_v, accn.at[nidx_v.at[k]], sem_n, add=True)
            pltpu.async_copy(ones_v, acce.at[eidx_v.at[k]], sem_e, add=True)
            return carry

        lax.fori_loop(0, NCHUNK, body, 0)
        pltpu.make_async_copy(ones_v, accn.at[nidx_v.at[0]], sem_n).wait()
        pltpu.make_async_copy(ones_v, acce.at[eidx_v.at[0]], sem_e).wait()
        plsc.subcore_barrier()
        pltpu.sync_copy(accn.at[pl.ds(s * OROWS, OROWS)],
                        out_hbm.at[c, 0, pl.ds(s * OROWS, OROWS)])
        pltpu.sync_copy(acce.at[pl.ds(s * OROWS, OROWS)],
                        out_hbm.at[c, 1, pl.ds(s * OROWS, OROWS)])

    return cnt


def _row_mask():
    ridx = lax.broadcasted_iota(jnp.int32, (ACC_ROWS, 1), 0)
    return ridx < N_NODES


def _bn(t, g, be):
    mask = _row_mask()
    tm = jnp.where(mask, t, 0.0)
    mu = jnp.sum(tm, axis=0, keepdims=True) / N_NODES
    dev = jnp.where(mask, t - mu, 0.0)
    var = jnp.sum(dev * dev, axis=0, keepdims=True) / N_NODES
    return g * (t - mu) / jnp.sqrt(var + EPS) + be


def _mm_body(x_ref, w_ref, o_ref):
    o_ref[...] = jnp.dot(x_ref[...], w_ref[...],
                         preferred_element_type=jnp.float32)


def _tc_mm(x, w):
    return pl.pallas_call(
        _mm_body,
        out_shape=jax.ShapeDtypeStruct((x.shape[0], w.shape[1]), jnp.float32),
    )(x, w)


def _scale_body(p_ref, cb0_ref, cb1_ref, o_ref):
    b = (cb0_ref[...] + cb1_ref[...])[:, 0:1]
    binv = jnp.where(b > 0, 1.0 / b, 0.0)[None]
    o_ref[...] = binv * (p_ref[0] + p_ref[1])


def _tc_scale(p, cb0, cb1):
    return pl.pallas_call(
        _scale_body,
        out_shape=jax.ShapeDtypeStruct(p.shape[1:], jnp.float32),
    )(p, cb0, cb1)


def _dinv_comb(q_ref, cd0_ref, cd1_ref):
    d = (cd0_ref[...] + cd1_ref[...])[:, 0:1]
    dinv = jnp.where(d > 0, 1.0 / d, 0.0)[None]
    qs = dinv * (q_ref[0] + q_ref[1])
    if qs.shape[0] == 1:
        return qs[0]
    return jnp.concatenate([qs[0], qs[1]], axis=1)


def _post_body(q_ref, cd0_ref, cd1_ref, b_ref, g_ref, be_ref, w_ref, o_ref):
    t = _dinv_comb(q_ref, cd0_ref, cd1_ref) + b_ref[...]
    h = jnp.maximum(_bn(t, g_ref[...], be_ref[...]), 0.0)
    r = jnp.dot(h, w_ref[...], preferred_element_type=jnp.float32)
    for hh in range(o_ref.shape[0]):
        o_ref[hh] = r[:, hh * FW:(hh + 1) * FW]


def _tc_post(q, cd0, cd1, b, g, be, w):
    hout = w.shape[1] // FW
    return pl.pallas_call(
        _post_body,
        out_shape=jax.ShapeDtypeStruct((hout, ACC_ROWS, FW), jnp.float32),
    )(q, cd0, cd1, b.reshape(1, -1), g.reshape(1, -1), be.reshape(1, -1), w)


def _head_body(q_ref, cd0_ref, cd1_ref, b_ref, g_ref, be_ref, bt_ref,
               wf1_ref, bf1_ref, wf2_ref, bf2_ref, o_ref):
    t = _dinv_comb(q_ref, cd0_ref, cd1_ref) + b_ref[...]
    h = _bn(t, g_ref[...], be_ref[...])
    te2 = jnp.maximum(bt_ref[...], 0.0) * TOPO_W
    row = jnp.dot(te2, wf1_ref[64:128, :],
                  preferred_element_type=jnp.float32)
    o = jnp.dot(h, wf1_ref[0:64, :],
                preferred_element_type=jnp.float32) + row + bf1_ref[...]
    o = jnp.maximum(o, 0.0)
    lg = jnp.dot(o, wf2_ref[...], preferred_element_type=jnp.float32)
    lg = lg + bf2_ref[...]
    m = jnp.max(lg, axis=1, keepdims=True)
    z = lg - m
    lse = jnp.log(jnp.sum(jnp.exp(z), axis=1, keepdims=True))
    o_ref[...] = (z - lse)[0:N_NODES]


def _tc_head(q, cd0, cd1, b, g, be, bt, wf1, bf1, wf2, bf2):
    return pl.pallas_call(
        _head_body,
        out_shape=jax.ShapeDtypeStruct((N_NODES, wf2.shape[1]), jnp.float32),
    )(q, cd0, cd1, b.reshape(1, -1), g.reshape(1, -1), be.reshape(1, -1),
      bt.reshape(1, -1), wf1, bf1.reshape(1, -1), wf2, bf2.reshape(1, -1))


def kernel(x, edge_index, W1, b1, g1, be1, W2, b2, g2, be2, W3, b3, g3, be3,
           Wt, bt, Wf1, bf1, Wf2, bf2):
    node = edge_index[0].astype(jnp.int32)
    he = edge_index[1].astype(jnp.int32)

    npad = NW * CHUNK * NCHUNK - N_INC
    spread = jnp.arange(npad, dtype=jnp.int32)

    def layout(idx, padvals):
        full = jnp.concatenate([idx, padvals])
        return full.reshape(NW, NCHUNK, CHUNK)

    src_pad = spread % N_NODES
    dst_pad = N_NODES + spread % NDUMP
    node_src = layout(node, src_pad)
    node_dst = layout(node, dst_pad)
    he_src = layout(he, src_pad)
    he_dst = layout(he, dst_pad)

    z64 = jnp.zeros((ACC_ROWS, FW), jnp.float32)
    zc = jnp.zeros((ACC_ROWS, CNT_W), jnp.float32)
    ones = jnp.ones((CHUNK, CNT_W), jnp.float32)

    seg1 = _make_seg_kernel(1)
    seg2 = _make_seg_kernel(2)
    cntk = _make_cnt_kernel()

    cnt = cntk(node_dst, he_dst, ones, zc)
    cd0, cd1 = cnt[0, 0], cnt[1, 0]
    cb0, cb1 = cnt[0, 1], cnt[1, 1]

    x_p = jnp.concatenate(
        [x, jnp.zeros((ACC_ROWS - N_NODES, x.shape[1]), jnp.float32)])

    xw = _tc_mm(x_p, W1)[None]
    p = seg1(xw, node_src, he_dst, z64)
    t = _tc_scale(p, cb0, cb1)
    q = seg1(t, he_src, node_dst, z64)
    xw = _tc_post(q, cd0, cd1, b1, g1, be1, W2)

    p = seg2(xw, node_src, he_dst, z64)
    t = _tc_scale(p, cb0, cb1)
    q = seg2(t, he_src, node_dst, z64)
    xw = _tc_post(q, cd0, cd1, b2, g2, be2, W3)

    p = seg1(xw, node_src, he_dst, z64)
    t = _tc_scale(p, cb0, cb1)
    q = seg1(t, he_src, node_dst, z64)

    return _tc_head(q, cd0, cd1, b3, g3, be3, bt, Wf1, bf1, Wf2, bf2)


# speedup vs baseline: 17.9341x; 1.1154x over previous
import functools

import jax
import jax.numpy as jnp
from jax import lax
from jax.experimental import pallas as pl
from jax.experimental.pallas import tpu as pltpu
from jax.experimental.pallas import tpu_sc as plsc

N_NODES = 10000
N_HE = 10000
N_INC = 320000
EPS = 1e-5
TOPO_W = 2.0

NCORE = 2
NSUB = 16
NW = NCORE * NSUB
CHUNK = 128
NCHUNK = -(-N_INC // (NW * CHUNK))
ZROWS = 632
ACC_ROWS = ZROWS * NSUB
NDUMP = ACC_ROWS - N_NODES
OROWS = ZROWS
CNT_W = 16
FW = 64
DEPTH = 3


def _make_seg_kernel(H):
    mesh = plsc.VectorSubcoreMesh(core_axis_name="c", subcore_axis_name="s")
    ngroups = -(-NCHUNK // DEPTH)

    @functools.partial(
        pl.kernel,
        mesh=mesh,
        out_type=jax.ShapeDtypeStruct((NCORE, H, ACC_ROWS, FW), jnp.float32),
        compiler_params=pltpu.CompilerParams(use_tc_tiling_on_sc=False),
        scratch_types=[
            pltpu.VMEM((NCHUNK, CHUNK), jnp.int32),
            pltpu.VMEM((NCHUNK, CHUNK), jnp.int32),
        ] + [pltpu.VMEM((CHUNK, FW), jnp.float32)] * DEPTH + [
            pltpu.VMEM_SHARED((ACC_ROWS, FW), jnp.float32),
            pltpu.VMEM_SHARED((ACC_ROWS, FW), jnp.float32),
        ] + [pltpu.SemaphoreType.DMA] * (2 * DEPTH),
    )
    def seg(table_hbm, src_hbm, dst_hbm, zeros_hbm, out_hbm,
            src_v, dst_v, *rest):
        rows = rest[:DEPTH]
        tbl = rest[DEPTH]
        acc = rest[DEPTH + 1]
        gsems = rest[DEPTH + 2:2 * DEPTH + 2]
        ssems = rest[2 * DEPTH + 2:3 * DEPTH + 2]
        c = lax.axis_index("c")
        s = lax.axis_index("s")
        wid = c * NSUB + s
        pltpu.sync_copy(src_hbm.at[wid], src_v)
        pltpu.sync_copy(dst_hbm.at[wid], dst_v)

        for h in range(H):
            pltpu.sync_copy(table_hbm.at[h, pl.ds(s * ZROWS, ZROWS)],
                            tbl.at[pl.ds(s * ZROWS, ZROWS)])
            pltpu.sync_copy(zeros_hbm.at[pl.ds(s * ZROWS, ZROWS)],
                            acc.at[pl.ds(s * ZROWS, ZROWS)])
            plsc.subcore_barrier()

            for b in range(DEPTH - 1):
                pltpu.async_copy(tbl.at[src_v.at[b]], rows[b], gsems[b])

            def body(g, carry):
                kb = g * DEPTH
                for b in range(DEPTH):
                    k = kb + b
                    bprev = (b - 1) % DEPTH

                    @pl.when(k < NCHUNK)
                    def _(k=k, b=b, bprev=bprev):
                        pltpu.make_async_copy(tbl.at[src_v.at[k]],
                                              rows[b], gsems[b]).wait()
                        pltpu.async_copy(rows[b], acc.at[dst_v.at[k]],
                                         ssems[b], add=True)

                        @pl.when(k + DEPTH - 1 < NCHUNK)
                        def _():
                            @pl.when(k > 0)
                            def _():
                                pltpu.make_async_copy(
                                    rows[bprev], acc.at[dst_v.at[0]],
                                    ssems[bprev]).wait()
                            pltpu.async_copy(tbl.at[src_v.at[k + DEPTH - 1]],
                                             rows[bprev], gsems[bprev])
                return carry

            lax.fori_loop(0, ngroups, body, 0)
            for b in range(DEPTH):
                pltpu.make_async_copy(rows[b], acc.at[dst_v.at[0]],
                                      ssems[b]).wait()
            plsc.subcore_barrier()
            pltpu.sync_copy(acc.at[pl.ds(s * OROWS, OROWS)],
                            out_hbm.at[c, h, pl.ds(s * OROWS, OROWS)])

    return seg


def _make_cnt_kernel():
    mesh = plsc.VectorSubcoreMesh(core_axis_name="c", subcore_axis_name="s")

    @functools.partial(
        pl.kernel,
        mesh=mesh,
        out_type=jax.ShapeDtypeStruct((NCORE, 2, ACC_ROWS, CNT_W), jnp.float32),
        compiler_params=pltpu.CompilerParams(use_tc_tiling_on_sc=False),
        scratch_types=[
            pltpu.VMEM((NCHUNK, CHUNK), jnp.int32),
            pltpu.VMEM((NCHUNK, CHUNK), jnp.int32),
            pltpu.VMEM((CHUNK, CNT_W), jnp.float32),
            pltpu.VMEM_SHARED((ACC_ROWS, CNT_W), jnp.float32),
            pltpu.VMEM_SHARED((ACC_ROWS, CNT_W), jnp.float32),
            pltpu.SemaphoreType.DMA,
            pltpu.SemaphoreType.DMA,
        ],
    )
    def cnt(nidx_hbm, eidx_hbm, ones_hbm, zeros_hbm, out_hbm,
            nidx_v, eidx_v, ones_v, accn, acce, sem_n, sem_e):
        c = lax.axis_index("c")
        s = lax.axis_index("s")
        wid = c * NSUB + s
        pltpu.sync_copy(zeros_hbm.at[pl.ds(s * ZROWS, ZROWS)],
                        accn.at[pl.ds(s * ZROWS, ZROWS)])
        pltpu.sync_copy(zeros_hbm.at[pl.ds(s * ZROWS, ZROWS)],
                        acce.at[pl.ds(s * ZROWS, ZROWS)])
        pltpu.sync_copy(ones_hbm, ones_v)
        pltpu.sync_copy(nidx_hbm.at[wid], nidx_v)
        pltpu.sync_copy(eidx_hbm.at[wid], eidx_v)
        plsc.subcore_barrier()

        def body(k, carry):
            @pl.when(k > 0)
            def _():
                pltpu.make_async_copy(ones_v, accn.at[nidx_v.at[0]],
                                      sem_n).wait()
                pltpu.make_async_copy(ones_v, acce.at[eidx_v.at[0]],
                                      sem_e).wait()
            pltpu.async_copy(ones_v, accn.at[nidx_v.at[k]], sem_n, add=True)
            pltpu.async_copy(ones_v, acce.at[eidx_v.at[k]], sem_e, add=True)
            return carry

        lax.fori_loop(0, NCHUNK, body, 0)
        pltpu.make_async_copy(ones_v, accn.at[nidx_v.at[0]], sem_n).wait()
        pltpu.make_async_copy(ones_v, acce.at[eidx_v.at[0]], sem_e).wait()
        plsc.subcore_barrier()
        pltpu.sync_copy(accn.at[pl.ds(s * OROWS, OROWS)],
                        out_hbm.at[c, 0, pl.ds(s * OROWS, OROWS)])
        pltpu.sync_copy(acce.at[pl.ds(s * OROWS, OROWS)],
                        out_hbm.at[c, 1, pl.ds(s * OROWS, OROWS)])

    return cnt


def _row_mask():
    ridx = lax.broadcasted_iota(jnp.int32, (ACC_ROWS, 1), 0)
    return ridx < N_NODES


def _bn(t, g, be):
    mask = _row_mask()
    tm = jnp.where(mask, t, 0.0)
    mu = jnp.sum(tm, axis=0, keepdims=True) / N_NODES
    dev = jnp.where(mask, t - mu, 0.0)
    var = jnp.sum(dev * dev, axis=0, keepdims=True) / N_NODES
    return g * (t - mu) / jnp.sqrt(var + EPS) + be


def _mm_body(x_ref, w_ref, o_ref):
    o_ref[...] = jnp.dot(x_ref[...], w_ref[...],
                         preferred_element_type=jnp.float32)


def _tc_mm(x, w):
    return pl.pallas_call(
        _mm_body,
        out_shape=jax.ShapeDtypeStruct((x.shape[0], w.shape[1]), jnp.float32),
    )(x, w)


def _scale_body(p_ref, cb0_ref, cb1_ref, o_ref):
    b = (cb0_ref[...] + cb1_ref[...])[:, 0:1]
    binv = jnp.where(b > 0, 1.0 / b, 0.0)[None]
    o_ref[...] = binv * (p_ref[0] + p_ref[1])


def _tc_scale(p, cb0, cb1):
    return pl.pallas_call(
        _scale_body,
        out_shape=jax.ShapeDtypeStruct(p.shape[1:], jnp.float32),
    )(p, cb0, cb1)


def _dinv_comb(q_ref, cd0_ref, cd1_ref):
    d = (cd0_ref[...] + cd1_ref[...])[:, 0:1]
    dinv = jnp.where(d > 0, 1.0 / d, 0.0)[None]
    qs = dinv * (q_ref[0] + q_ref[1])
    if qs.shape[0] == 1:
        return qs[0]
    return jnp.concatenate([qs[0], qs[1]], axis=1)


def _post_body(q_ref, cd0_ref, cd1_ref, b_ref, g_ref, be_ref, w_ref, o_ref):
    t = _dinv_comb(q_ref, cd0_ref, cd1_ref) + b_ref[...]
    h = jnp.maximum(_bn(t, g_ref[...], be_ref[...]), 0.0)
    r = jnp.dot(h, w_ref[...], preferred_element_type=jnp.float32)
    for hh in range(o_ref.shape[0]):
        o_ref[hh] = r[:, hh * FW:(hh + 1) * FW]


def _tc_post(q, cd0, cd1, b, g, be, w):
    hout = w.shape[1] // FW
    return pl.pallas_call(
        _post_body,
        out_shape=jax.ShapeDtypeStruct((hout, ACC_ROWS, FW), jnp.float32),
    )(q, cd0, cd1, b.reshape(1, -1), g.reshape(1, -1), be.reshape(1, -1), w)


def _head_body(q_ref, cd0_ref, cd1_ref, b_ref, g_ref, be_ref, bt_ref,
               wf1_ref, bf1_ref, wf2_ref, bf2_ref, o_ref):
    t = _dinv_comb(q_ref, cd0_ref, cd1_ref) + b_ref[...]
    h = _bn(t, g_ref[...], be_ref[...])
    te2 = jnp.maximum(bt_ref[...], 0.0) * TOPO_W
    row = jnp.dot(te2, wf1_ref[64:128, :],
                  preferred_element_type=jnp.float32)
    o = jnp.dot(h, wf1_ref[0:64, :],
                preferred_element_type=jnp.float32) + row + bf1_ref[...]
    o = jnp.maximum(o, 0.0)
    lg = jnp.dot(o, wf2_ref[...], preferred_element_type=jnp.float32)
    lg = lg + bf2_ref[...]
    m = jnp.max(lg, axis=1, keepdims=True)
    z = lg - m
    lse = jnp.log(jnp.sum(jnp.exp(z), axis=1, keepdims=True))
    o_ref[...] = (z - lse)[0:N_NODES]


def _tc_head(q, cd0, cd1, b, g, be, bt, wf1, bf1, wf2, bf2):
    return pl.pallas_call(
        _head_body,
        out_shape=jax.ShapeDtypeStruct((N_NODES, wf2.shape[1]), jnp.float32),
    )(q, cd0, cd1, b.reshape(1, -1), g.reshape(1, -1), be.reshape(1, -1),
      bt.reshape(1, -1), wf1, bf1.reshape(1, -1), wf2, bf2.reshape(1, -1))


def kernel(x, edge_index, W1, b1, g1, be1, W2, b2, g2, be2, W3, b3, g3, be3,
           Wt, bt, Wf1, bf1, Wf2, bf2):
    node = edge_index[0].astype(jnp.int32)
    he = edge_index[1].astype(jnp.int32)

    npad = NW * CHUNK * NCHUNK - N_INC
    spread = jnp.arange(npad, dtype=jnp.int32)

    def layout(idx, padvals):
        full = jnp.concatenate([idx, padvals])
        return full.reshape(NW, NCHUNK, CHUNK)

    src_pad = spread % N_NODES
    dst_pad = N_NODES + spread % NDUMP
    node_src = layout(node, src_pad)
    node_dst = layout(node, dst_pad)
    he_src = layout(he, src_pad)
    he_dst = layout(he, dst_pad)

    z64 = jnp.zeros((ACC_ROWS, FW), jnp.float32)
    zc = jnp.zeros((ACC_ROWS, CNT_W), jnp.float32)
    ones = jnp.ones((CHUNK, CNT_W), jnp.float32)

    seg1 = _make_seg_kernel(1)
    seg2 = _make_seg_kernel(2)
    cntk = _make_cnt_kernel()

    cnt = cntk(node_dst, he_dst, ones, zc)
    cd0, cd1 = cnt[0, 0], cnt[1, 0]
    cb0, cb1 = cnt[0, 1], cnt[1, 1]

    x_p = jnp.concatenate(
        [x, jnp.zeros((ACC_ROWS - N_NODES, x.shape[1]), jnp.float32)])

    xw = _tc_mm(x_p, W1)[None]
    p = seg1(xw, node_src, he_dst, z64)
    t = _tc_scale(p, cb0, cb1)
    q = seg1(t, he_src, node_dst, z64)
    xw = _tc_post(q, cd0, cd1, b1, g1, be1, W2)

    p = seg2(xw, node_src, he_dst, z64)
    t = _tc_scale(p, cb0, cb1)
    q = seg2(t, he_src, node_dst, z64)
    xw = _tc_post(q, cd0, cd1, b2, g2, be2, W3)

    p = seg1(xw, node_src, he_dst, z64)
    t = _tc_scale(p, cb0, cb1)
    q = seg1(t, he_src, node_dst, z64)

    return _tc_head(q, cd0, cd1, b3, g3, be3, bt, Wf1, bf1, Wf2, bf2)


# speedup vs baseline: 18.0410x; 1.0060x over previous
import functools

import jax
import jax.numpy as jnp
from jax import lax
from jax.experimental import pallas as pl
from jax.experimental.pallas import tpu as pltpu
from jax.experimental.pallas import tpu_sc as plsc

N_NODES = 10000
N_HE = 10000
N_INC = 320000
EPS = 1e-5
TOPO_W = 2.0

NCORE = 2
NSUB = 16
NW = NCORE * NSUB
CHUNK = 128
NCHUNK = -(-N_INC // (NW * CHUNK))
ZROWS = 632
ACC_ROWS = ZROWS * NSUB
NDUMP = ACC_ROWS - N_NODES
OROWS = ZROWS
CNT_W = 16
FW = 64
DEPTH = 3
CLAG = 8


def _make_seg_kernel(H):
    mesh = plsc.VectorSubcoreMesh(core_axis_name="c", subcore_axis_name="s")
    ngroups = -(-NCHUNK // DEPTH)

    @functools.partial(
        pl.kernel,
        mesh=mesh,
        out_type=jax.ShapeDtypeStruct((NCORE, H, ACC_ROWS, FW), jnp.float32),
        compiler_params=pltpu.CompilerParams(use_tc_tiling_on_sc=False),
        scratch_types=[
            pltpu.VMEM((NCHUNK, CHUNK), jnp.int32),
            pltpu.VMEM((NCHUNK, CHUNK), jnp.int32),
        ] + [pltpu.VMEM((CHUNK, FW), jnp.float32)] * DEPTH + [
            pltpu.VMEM_SHARED((ACC_ROWS, FW), jnp.float32),
            pltpu.VMEM_SHARED((ACC_ROWS, FW), jnp.float32),
        ] + [pltpu.SemaphoreType.DMA] * (2 * DEPTH),
    )
    def seg(table_hbm, src_hbm, dst_hbm, zeros_hbm, out_hbm,
            src_v, dst_v, *rest):
        rows = rest[:DEPTH]
        tbl = rest[DEPTH]
        acc = rest[DEPTH + 1]
        gsems = rest[DEPTH + 2:2 * DEPTH + 2]
        ssems = rest[2 * DEPTH + 2:3 * DEPTH + 2]
        c = lax.axis_index("c")
        s = lax.axis_index("s")
        wid = c * NSUB + s
        pltpu.sync_copy(src_hbm.at[wid], src_v)
        pltpu.sync_copy(dst_hbm.at[wid], dst_v)

        for h in range(H):
            pltpu.sync_copy(table_hbm.at[h, pl.ds(s * ZROWS, ZROWS)],
                            tbl.at[pl.ds(s * ZROWS, ZROWS)])
            pltpu.sync_copy(zeros_hbm.at[pl.ds(s * ZROWS, ZROWS)],
                            acc.at[pl.ds(s * ZROWS, ZROWS)])
            plsc.subcore_barrier()

            for b in range(DEPTH - 1):
                pltpu.async_copy(tbl.at[src_v.at[b]], rows[b], gsems[b])

            def body(g, carry):
                kb = g * DEPTH
                for b in range(DEPTH):
                    k = kb + b
                    bprev = (b - 1) % DEPTH

                    @pl.when(k < NCHUNK)
                    def _(k=k, b=b, bprev=bprev):
                        pltpu.make_async_copy(tbl.at[src_v.at[k]],
                                              rows[b], gsems[b]).wait()
                        pltpu.async_copy(rows[b], acc.at[dst_v.at[k]],
                                         ssems[b], add=True)

                        @pl.when(k + DEPTH - 1 < NCHUNK)
                        def _():
                            @pl.when(k > 0)
                            def _():
                                pltpu.make_async_copy(
                                    rows[bprev], acc.at[dst_v.at[0]],
                                    ssems[bprev]).wait()
                            pltpu.async_copy(tbl.at[src_v.at[k + DEPTH - 1]],
                                             rows[bprev], gsems[bprev])
                return carry

            lax.fori_loop(0, ngroups, body, 0)
            for b in range(DEPTH):
                pltpu.make_async_copy(rows[b], acc.at[dst_v.at[0]],
                                      ssems[b]).wait()
            plsc.subcore_barrier()
            pltpu.sync_copy(acc.at[pl.ds(s * OROWS, OROWS)],
                            out_hbm.at[c, h, pl.ds(s * OROWS, OROWS)])

    return seg


def _make_cnt_kernel():
    mesh = plsc.VectorSubcoreMesh(core_axis_name="c", subcore_axis_name="s")

    @functools.partial(
        pl.kernel,
        mesh=mesh,
        out_type=jax.ShapeDtypeStruct((NCORE, 2, ACC_ROWS, CNT_W), jnp.float32),
        compiler_params=pltpu.CompilerParams(use_tc_tiling_on_sc=False),
        scratch_types=[
            pltpu.VMEM((NCHUNK, CHUNK), jnp.int32),
            pltpu.VMEM((NCHUNK, CHUNK), jnp.int32),
            pltpu.VMEM((CHUNK, CNT_W), jnp.float32),
            pltpu.VMEM_SHARED((ACC_ROWS, CNT_W), jnp.float32),
            pltpu.VMEM_SHARED((ACC_ROWS, CNT_W), jnp.float32),
            pltpu.SemaphoreType.DMA,
            pltpu.SemaphoreType.DMA,
        ],
    )
    def cnt(nidx_hbm, eidx_hbm, ones_hbm, zeros_hbm, out_hbm,
            nidx_v, eidx_v, ones_v, accn, acce, sem_n, sem_e):
        c = lax.axis_index("c")
        s = lax.axis_index("s")
        wid = c * NSUB + s
        pltpu.sync_copy(zeros_hbm.at[pl.ds(s * ZROWS, ZROWS)],
                        accn.at[pl.ds(s * ZROWS, ZROWS)])
        pltpu.sync_copy(zeros_hbm.at[pl.ds(s * ZROWS, ZROWS)],
                        acce.at[pl.ds(s * ZROWS, ZROWS)])
        pltpu.sync_copy(ones_hbm, ones_v)
        pltpu.sync_copy(nidx_hbm.at[wid], nidx_v)
        pltpu.sync_copy(eidx_hbm.at[wid], eidx_v)
        plsc.subcore_barrier()

        def body(k, carry):
            @pl.when(k >= CLAG)
            def _():
                pltpu.make_async_copy(ones_v, accn.at[nidx_v.at[0]],
                                      sem_n).wait()
                pltpu.make_async_copy(ones_v, acce.at[eidx_v.at[0]],
                                      sem_e).wait()
            pltpu.async_copy(ones_v, accn.at[nidx_v.at[k]], sem_n, add=True)
            pltpu.async_copy(ones_v, acce.at[eidx_v.at[k]], sem_e, add=True)
            return carry

        lax.fori_loop(0, NCHUNK, body, 0)
        for _i in range(CLAG):
            pltpu.make_async_copy(ones_v, accn.at[nidx_v.at[0]], sem_n).wait()
            pltpu.make_async_copy(ones_v, acce.at[eidx_v.at[0]], sem_e).wait()
        plsc.subcore_barrier()
        pltpu.sync_copy(accn.at[pl.ds(s * OROWS, OROWS)],
                        out_hbm.at[c, 0, pl.ds(s * OROWS, OROWS)])
        pltpu.sync_copy(acce.at[pl.ds(s * OROWS, OROWS)],
                        out_hbm.at[c, 1, pl.ds(s * OROWS, OROWS)])

    return cnt


def _row_mask():
    ridx = lax.broadcasted_iota(jnp.int32, (ACC_ROWS, 1), 0)
    return ridx < N_NODES


def _bn(t, g, be):
    mask = _row_mask()
    tm = jnp.where(mask, t, 0.0)
    mu = jnp.sum(tm, axis=0, keepdims=True) / N_NODES
    dev = jnp.where(mask, t - mu, 0.0)
    var = jnp.sum(dev * dev, axis=0, keepdims=True) / N_NODES
    return g * (t - mu) / jnp.sqrt(var + EPS) + be


def _mm_body(x_ref, w_ref, o_ref):
    o_ref[...] = jnp.dot(x_ref[...], w_ref[...],
                         preferred_element_type=jnp.float32)


def _tc_mm(x, w):
    return pl.pallas_call(
        _mm_body,
        out_shape=jax.ShapeDtypeStruct((x.shape[0], w.shape[1]), jnp.float32),
    )(x, w)


def _scale_body(p_ref, cb0_ref, cb1_ref, o_ref):
    b = (cb0_ref[...] + cb1_ref[...])[:, 0:1]
    binv = jnp.where(b > 0, 1.0 / b, 0.0)[None]
    o_ref[...] = binv * (p_ref[0] + p_ref[1])


def _tc_scale(p, cb0, cb1):
    return pl.pallas_call(
        _scale_body,
        out_shape=jax.ShapeDtypeStruct(p.shape[1:], jnp.float32),
    )(p, cb0, cb1)


def _dinv_comb(q_ref, cd0_ref, cd1_ref):
    d = (cd0_ref[...] + cd1_ref[...])[:, 0:1]
    dinv = jnp.where(d > 0, 1.0 / d, 0.0)[None]
    qs = dinv * (q_ref[0] + q_ref[1])
    if qs.shape[0] == 1:
        return qs[0]
    return jnp.concatenate([qs[0], qs[1]], axis=1)


def _post_body(q_ref, cd0_ref, cd1_ref, b_ref, g_ref, be_ref, w_ref, o_ref):
    t = _dinv_comb(q_ref, cd0_ref, cd1_ref) + b_ref[...]
    h = jnp.maximum(_bn(t, g_ref[...], be_ref[...]), 0.0)
    r = jnp.dot(h, w_ref[...], preferred_element_type=jnp.float32)
    for hh in range(o_ref.shape[0]):
        o_ref[hh] = r[:, hh * FW:(hh + 1) * FW]


def _tc_post(q, cd0, cd1, b, g, be, w):
    hout = w.shape[1] // FW
    return pl.pallas_call(
        _post_body,
        out_shape=jax.ShapeDtypeStruct((hout, ACC_ROWS, FW), jnp.float32),
    )(q, cd0, cd1, b.reshape(1, -1), g.reshape(1, -1), be.reshape(1, -1), w)


def _head_body(q_ref, cd0_ref, cd1_ref, b_ref, g_ref, be_ref, bt_ref,
               wf1_ref, bf1_ref, wf2_ref, bf2_ref, o_ref):
    t = _dinv_comb(q_ref, cd0_ref, cd1_ref) + b_ref[...]
    h = _bn(t, g_ref[...], be_ref[...])
    te2 = jnp.maximum(bt_ref[...], 0.0) * TOPO_W
    row = jnp.dot(te2, wf1_ref[64:128, :],
                  preferred_element_type=jnp.float32)
    o = jnp.dot(h, wf1_ref[0:64, :],
                preferred_element_type=jnp.float32) + row + bf1_ref[...]
    o = jnp.maximum(o, 0.0)
    lg = jnp.dot(o, wf2_ref[...], preferred_element_type=jnp.float32)
    lg = lg + bf2_ref[...]
    m = jnp.max(lg, axis=1, keepdims=True)
    z = lg - m
    lse = jnp.log(jnp.sum(jnp.exp(z), axis=1, keepdims=True))
    o_ref[...] = (z - lse)[0:N_NODES]


def _tc_head(q, cd0, cd1, b, g, be, bt, wf1, bf1, wf2, bf2):
    return pl.pallas_call(
        _head_body,
        out_shape=jax.ShapeDtypeStruct((N_NODES, wf2.shape[1]), jnp.float32),
    )(q, cd0, cd1, b.reshape(1, -1), g.reshape(1, -1), be.reshape(1, -1),
      bt.reshape(1, -1), wf1, bf1.reshape(1, -1), wf2, bf2.reshape(1, -1))


def kernel(x, edge_index, W1, b1, g1, be1, W2, b2, g2, be2, W3, b3, g3, be3,
           Wt, bt, Wf1, bf1, Wf2, bf2):
    node = edge_index[0].astype(jnp.int32)
    he = edge_index[1].astype(jnp.int32)

    npad = NW * CHUNK * NCHUNK - N_INC
    spread = jnp.arange(npad, dtype=jnp.int32)

    def layout(idx, padvals):
        full = jnp.concatenate([idx, padvals])
        return full.reshape(NW, NCHUNK, CHUNK)

    src_pad = spread % N_NODES
    dst_pad = N_NODES + spread % NDUMP
    node_src = layout(node, src_pad)
    node_dst = layout(node, dst_pad)
    he_src = layout(he, src_pad)
    he_dst = layout(he, dst_pad)

    z64 = jnp.zeros((ACC_ROWS, FW), jnp.float32)
    zc = jnp.zeros((ACC_ROWS, CNT_W), jnp.float32)
    ones = jnp.ones((CHUNK, CNT_W), jnp.float32)

    seg1 = _make_seg_kernel(1)
    seg2 = _make_seg_kernel(2)
    cntk = _make_cnt_kernel()

    cnt = cntk(node_dst, he_dst, ones, zc)
    cd0, cd1 = cnt[0, 0], cnt[1, 0]
    cb0, cb1 = cnt[0, 1], cnt[1, 1]

    x_p = jnp.concatenate(
        [x, jnp.zeros((ACC_ROWS - N_NODES, x.shape[1]), jnp.float32)])

    xw = _tc_mm(x_p, W1)[None]
    p = seg1(xw, node_src, he_dst, z64)
    t = _tc_scale(p, cb0, cb1)
    q = seg1(t, he_src, node_dst, z64)
    xw = _tc_post(q, cd0, cd1, b1, g1, be1, W2)

    p = seg2(xw, node_src, he_dst, z64)
    t = _tc_scale(p, cb0, cb1)
    q = seg2(t, he_src, node_dst, z64)
    xw = _tc_post(q, cd0, cd1, b2, g2, be2, W3)

    p = seg1(xw, node_src, he_dst, z64)
    t = _tc_scale(p, cb0, cb1)
    q = seg1(t, he_src, node_dst, z64)

    return _tc_head(q, cd0, cd1, b3, g3, be3, bt, Wf1, bf1, Wf2, bf2)
